# baseline (speedup 1.0000x reference)
import functools

import jax
import jax.numpy as jnp
from jax import lax
from jax.experimental import pallas as pl
from jax.experimental.pallas import tpu as pltpu
from jax.experimental.pallas import tpu_sc as plsc

K = 256
L = 16


def _clear_hist(hist_ref, nwords):
    @plsc.parallel_loop(0, nwords // L, unroll=8)
    def body(i):
        hist_ref[pl.ds(i * L, L)] = jnp.zeros((L,), jnp.int32)


def _merge_hist(hist_ref, coarse_ref):
    zeros = jnp.zeros((L,), jnp.int32)

    @plsc.parallel_loop(0, 256 // L)
    def body(jb):
        acc = zeros
        for b_lo in range(L):
            acc = acc + hist_ref[pl.ds(b_lo * 256 + jb * L, L)]
        coarse_ref[pl.ds(jb * L, L)] = acc


def _scan_level(hist_ref, r):
    lanes = lax.iota(jnp.int32, L)

    def body(jj, carry):
        suffix, best_b, cge_b, h_b = carry
        j = 15 - jj
        v = hist_ref[pl.ds(j * L, L)]
        rv = lax.rev(v, (0,))
        cs = plsc.cumsum(rv) + suffix
        m = cs >= r
        npos = jnp.max(plsc.all_reduce_population_count(m))
        l = jnp.max(plsc.all_reduce_ffs(m))
        sel = lanes == l
        cs_l = jnp.max(jnp.where(sel, cs, 0))
        rv_l = jnp.max(jnp.where(sel, rv, 0))
        b = j * L + (L - 1) - l
        upd = jnp.logical_and(npos > 0, best_b < 0)
        best_b = jnp.where(upd, b, best_b)
        cge_b = jnp.where(upd, cs_l, cge_b)
        h_b = jnp.where(upd, rv_l, h_b)
        suffix = suffix + jnp.sum(v)
        return suffix, best_b, cge_b, h_b

    z32 = jnp.int32(0)
    _, best_b, cge_b, h_b = lax.fori_loop(
        0, 256 // L, body, (z32, jnp.int32(-1), z32, z32)
    )
    return best_b, r - (cge_b - h_b)


def _scan_fine(hist_ref, b_hi, r):
    lanes = lax.iota(jnp.int32, L)
    f = plsc.load_gather(hist_ref, [lanes * 256 + b_hi])
    rv = lax.rev(f, (0,))
    cs = plsc.cumsum(rv)
    m = cs >= r
    l = jnp.max(plsc.all_reduce_ffs(m))
    sel = lanes == l
    cs_l = jnp.max(jnp.where(sel, cs, 0))
    rv_l = jnp.max(jnp.where(sel, rv, 0))
    b_lo = (L - 1) - l
    return b_lo, r - (cs_l - rv_l)


def _row_select(row_ref, hist_ref, coarse_ref, buf_ref, nvec):
    ones = jnp.ones((L,), jnp.int32)

    @plsc.parallel_loop(0, nvec, unroll=8)
    def l0(i):
        v = row_ref[pl.ds(i * L, L)]
        z = jnp.maximum(lax.bitcast_convert_type(v, jnp.int32), 0)
        row_ref[pl.ds(i * L, L)] = lax.bitcast_convert_type(z, jnp.float32)
        b_hi = lax.shift_right_logical(z, 23)
        b_lo = lax.shift_right_logical(z, 19) & 15
        plsc.addupdate_scatter(hist_ref, [b_lo * 256 + b_hi], ones)

    _merge_hist(hist_ref, coarse_ref)
    b_hi, r = _scan_level(coarse_ref, jnp.int32(K))
    b_lo, r = _scan_fine(hist_ref, b_hi, r)
    _clear_hist(hist_ref, 4096)
    prefix = ((b_hi << 4) | b_lo) << 19
    p12 = lax.shift_right_logical(prefix, 19)

    @plsc.parallel_loop(0, nvec, carry=jnp.int32(0))
    def compact(i, off):
        z = lax.bitcast_convert_type(row_ref[pl.ds(i * L, L)], jnp.int32)
        m = lax.shift_right_logical(z, 19) == p12
        plsc.store_compressed(buf_ref.at[pl.ds(off, L)], z, mask=m)
        return off + jnp.max(plsc.all_reduce_population_count(m))

    off = compact
    buf_ref[pl.ds(off, L)] = jnp.zeros((L,), jnp.int32)

    nbufvec = lax.div(off + (L - 1), jnp.int32(L))

    def bit_step(bi, t):
        cand = t | (jnp.int32(1) << (18 - bi))

        def acc_body(i, acc):
            return acc + (buf_ref[pl.ds(i * L, L)] >= cand).astype(jnp.int32)

        acc = lax.fori_loop(0, nbufvec, acc_body, jnp.zeros((L,), jnp.int32))
        cnt = jnp.sum(acc)
        return jnp.where(cnt >= r, cand, t)

    t = lax.fori_loop(0, 19, bit_step, prefix)

    @plsc.parallel_loop(0, nvec, unroll=8)
    def out_body(i):
        zf = row_ref[pl.ds(i * L, L)]
        z = lax.bitcast_convert_type(zf, jnp.int32)
        row_ref[pl.ds(i * L, L)] = jnp.where(z >= t, zf, 0.0)


def kernel(x):
    m, n = x.shape
    nw = 32
    rows_per_w = m // nw
    nvec = n // L
    mesh = plsc.VectorSubcoreMesh(
        core_axis_name="c", subcore_axis_name="s", num_cores=2, num_subcores=16
    )

    @functools.partial(
        pl.kernel,
        out_type=jax.ShapeDtypeStruct((m, n), jnp.float32),
        mesh=mesh,
        scratch_types=[
            pltpu.VMEM((n,), jnp.float32),
            pltpu.VMEM((4096,), jnp.int32),
            pltpu.VMEM((256,), jnp.int32),
            pltpu.VMEM((n + L,), jnp.int32),
        ],
        compiler_params=pltpu.CompilerParams(needs_layout_passes=False),
    )
    def sc_kernel(x_hbm, out_hbm, row_v, hist_v, coarse_v, buf_v):
        wid = lax.axis_index("s") * 2 + lax.axis_index("c")
        _clear_hist(hist_v, 4096)
        for rr in range(rows_per_w):
            row = wid * rows_per_w + rr
            pltpu.sync_copy(x_hbm.at[row], row_v)
            _row_select(row_v, hist_v, coarse_v, buf_v, nvec)
            pltpu.sync_copy(row_v, out_hbm.at[row])

    return sc_kernel(x)


# speedup vs baseline: 1.0998x; 1.0998x over previous
import functools

import jax
import jax.numpy as jnp
from jax import lax
from jax.experimental import pallas as pl
from jax.experimental.pallas import tpu as pltpu
from jax.experimental.pallas import tpu_sc as plsc

K = 256
L = 16


def _clear(ref, nwords):
    @plsc.parallel_loop(0, nwords // L, unroll=8)
    def body(i):
        ref[pl.ds(i * L, L)] = jnp.zeros((L,), jnp.int32)


def _merge_hist(hist_ref, coarse_ref):
    zeros = jnp.zeros((L,), jnp.int32)

    @plsc.parallel_loop(0, 256 // L)
    def body(jb):
        acc = zeros
        for lo4 in range(L):
            acc = acc + hist_ref[pl.ds(lo4 * 256 + jb * L, L)]
        coarse_ref[pl.ds(jb * L, L)] = acc


def _scan_coarse(coarse_ref, nvec, r):
    lanes = lax.iota(jnp.int32, L)

    def body(jj, carry):
        suffix, best_b, cge_b, h_b = carry
        j = (nvec - 1) - jj
        v = coarse_ref[pl.ds(j * L, L)]
        rv = lax.rev(v, (0,))
        cs = plsc.cumsum(rv) + suffix
        m = cs >= r
        l = jnp.max(plsc.all_reduce_ffs(m))
        sel = lanes == l
        cs_l = jnp.max(jnp.where(sel, cs, 0))
        rv_l = jnp.max(jnp.where(sel, rv, 0))
        b = j * L + (L - 1) - l
        upd = jnp.logical_and(l < L, best_b < 0)
        best_b = jnp.where(upd, b, best_b)
        cge_b = jnp.where(upd, cs_l, cge_b)
        h_b = jnp.where(upd, rv_l, h_b)
        suffix = suffix + jnp.sum(v)
        return suffix, best_b, cge_b, h_b

    z32 = jnp.int32(0)
    _, best_b, cge_b, h_b = lax.fori_loop(
        0, nvec, body, (z32, jnp.int32(-1), z32, z32)
    )
    return best_b, r - (cge_b - h_b)


def _scan_fine(hist_ref, b_hi, r):
    lanes = lax.iota(jnp.int32, L)
    f = plsc.load_gather(hist_ref, [lanes * 256 + b_hi])
    rv = lax.rev(f, (0,))
    cs = plsc.cumsum(rv)
    m = cs >= r
    l = jnp.max(plsc.all_reduce_ffs(m))
    sel = lanes == l
    cs_l = jnp.max(jnp.where(sel, cs, 0))
    rv_l = jnp.max(jnp.where(sel, rv, 0))
    b_lo = (L - 1) - l
    return b_lo, r - (cs_l - rv_l)


def _row_select(row_ref, hist_ref, coarse_ref, nvec):
    ones = jnp.ones((L,), jnp.int32)

    @plsc.parallel_loop(0, nvec, unroll=8)
    def l0(i):
        v = row_ref[pl.ds(i * L, L)]
        z = jnp.maximum(lax.bitcast_convert_type(v, jnp.int32), 0)
        row_ref[pl.ds(i * L, L)] = lax.bitcast_convert_type(z, jnp.float32)
        hi8 = lax.shift_right_logical(z, 23)
        lo4 = lax.shift_right_logical(z, 19) & 15
        plsc.addupdate_scatter(hist_ref, [lo4 * 256 + hi8], ones)

    _merge_hist(hist_ref, coarse_ref)
    b_hi, r = _scan_coarse(coarse_ref, 16, jnp.int32(K))
    b_lo, r = _scan_fine(hist_ref, b_hi, r)
    _clear(hist_ref, 4096)
    prefix = ((b_hi << 4) | b_lo) << 19
    p12 = lax.shift_right_logical(prefix, 19)

    @plsc.parallel_loop(0, nvec, unroll=8)
    def l1(i):
        z = lax.bitcast_convert_type(row_ref[pl.ds(i * L, L)], jnp.int32)
        m = lax.shift_right_logical(z, 19) == p12
        hi8 = lax.shift_right_logical(z, 11) & 255
        lo4 = lax.shift_right_logical(z, 7) & 15
        plsc.addupdate_scatter(hist_ref, [lo4 * 256 + hi8], ones, mask=m)

    _merge_hist(hist_ref, coarse_ref)
    b_hi, r = _scan_coarse(coarse_ref, 16, r)
    b_lo, r = _scan_fine(hist_ref, b_hi, r)
    _clear(hist_ref, 4096)
    prefix = prefix | (((b_hi << 4) | b_lo) << 7)
    p24 = lax.shift_right_logical(prefix, 7)

    @plsc.parallel_loop(0, nvec, unroll=8)
    def l2(i):
        z = lax.bitcast_convert_type(row_ref[pl.ds(i * L, L)], jnp.int32)
        m = lax.shift_right_logical(z, 7) == p24
        plsc.addupdate_scatter(hist_ref, [z & 127], ones, mask=m)

    b7, _r = _scan_coarse(hist_ref, 8, r)
    _clear(hist_ref, 128)
    t = prefix | b7

    @plsc.parallel_loop(0, nvec, unroll=8)
    def out_body(i):
        zf = row_ref[pl.ds(i * L, L)]
        z = lax.bitcast_convert_type(zf, jnp.int32)
        row_ref[pl.ds(i * L, L)] = jnp.where(z >= t, zf, 0.0)


def kernel(x):
    m, n = x.shape
    nw = 32
    rows_per_w = m // nw
    nvec = n // L
    mesh = plsc.VectorSubcoreMesh(
        core_axis_name="c", subcore_axis_name="s", num_cores=2, num_subcores=16
    )

    @functools.partial(
        pl.kernel,
        out_type=jax.ShapeDtypeStruct((m, n), jnp.float32),
        mesh=mesh,
        scratch_types=[
            pltpu.VMEM((n,), jnp.float32),
            pltpu.VMEM((4096,), jnp.int32),
            pltpu.VMEM((256,), jnp.int32),
        ],
        compiler_params=pltpu.CompilerParams(needs_layout_passes=False),
    )
    def sc_kernel(x_hbm, out_hbm, row_v, hist_v, coarse_v):
        wid = lax.axis_index("s") * 2 + lax.axis_index("c")
        _clear(hist_v, 4096)
        for rr in range(rows_per_w):
            row = wid * rows_per_w + rr
            pltpu.sync_copy(x_hbm.at[row], row_v)
            _row_select(row_v, hist_v, coarse_v, nvec)
            pltpu.sync_copy(row_v, out_hbm.at[row])

    return sc_kernel(x)


# speedup vs baseline: 1.1630x; 1.0575x over previous
import functools

import jax
import jax.numpy as jnp
from jax import lax
from jax.experimental import pallas as pl
from jax.experimental.pallas import tpu as pltpu
from jax.experimental.pallas import tpu_sc as plsc

K = 256
L = 16


def _clear(ref, nwords):
    @plsc.parallel_loop(0, nwords // L, unroll=8)
    def body(i):
        ref[pl.ds(i * L, L)] = jnp.zeros((L,), jnp.int32)


def _merge_hist(hist_ref, coarse_ref):
    zeros = jnp.zeros((L,), jnp.int32)

    @plsc.parallel_loop(0, 256 // L)
    def body(jb):
        acc = zeros
        for lo4 in range(L):
            acc = acc + hist_ref[pl.ds(lo4 * 256 + jb * L, L)]
        coarse_ref[pl.ds(jb * L, L)] = acc


def _scan_coarse(coarse_ref, nvec, r):
    lanes = lax.iota(jnp.int32, L)

    def body(jj, carry):
        suffix, best_b, cge_b, h_b = carry
        j = (nvec - 1) - jj
        v = coarse_ref[pl.ds(j * L, L)]
        rv = lax.rev(v, (0,))
        cs = plsc.cumsum(rv) + suffix
        m = cs >= r
        l = jnp.max(plsc.all_reduce_ffs(m))
        sel = lanes == l
        cs_l = jnp.max(jnp.where(sel, cs, 0))
        rv_l = jnp.max(jnp.where(sel, rv, 0))
        b = j * L + (L - 1) - l
        upd = jnp.logical_and(l < L, best_b < 0)
        best_b = jnp.where(upd, b, best_b)
        cge_b = jnp.where(upd, cs_l, cge_b)
        h_b = jnp.where(upd, rv_l, h_b)
        suffix = suffix + jnp.sum(v)
        return suffix, best_b, cge_b, h_b

    z32 = jnp.int32(0)
    _, best_b, cge_b, h_b = lax.fori_loop(
        0, nvec, body, (z32, jnp.int32(-1), z32, z32)
    )
    return best_b, r - (cge_b - h_b)


def _scan_fine(hist_ref, b_hi, r):
    lanes = lax.iota(jnp.int32, L)
    f = plsc.load_gather(hist_ref, [lanes * 256 + b_hi])
    rv = lax.rev(f, (0,))
    cs = plsc.cumsum(rv)
    m = cs >= r
    l = jnp.max(plsc.all_reduce_ffs(m))
    sel = lanes == l
    cs_l = jnp.max(jnp.where(sel, cs, 0))
    rv_l = jnp.max(jnp.where(sel, rv, 0))
    b_lo = (L - 1) - l
    return b_lo, r - (cs_l - rv_l)


def _row_select(row_ref, hist_ref, coarse_ref, nvec):
    ones = jnp.ones((L,), jnp.int32)

    @plsc.parallel_loop(0, nvec, unroll=8)
    def l0(i):
        v = row_ref[pl.ds(i * L, L)]
        z = jnp.maximum(lax.bitcast_convert_type(v, jnp.int32), 0)
        row_ref[pl.ds(i * L, L)] = lax.bitcast_convert_type(z, jnp.float32)
        hi8 = lax.shift_right_logical(z, 23)
        lo4 = lax.shift_right_logical(z, 19) & 15
        plsc.addupdate_scatter(hist_ref, [lo4 * 256 + hi8], ones)

    _merge_hist(hist_ref, coarse_ref)
    b_hi, r = _scan_coarse(coarse_ref, 16, jnp.int32(K))
    b_lo, r = _scan_fine(hist_ref, b_hi, r)
    _clear(hist_ref, 4096)
    prefix = ((b_hi << 4) | b_lo) << 19
    p12 = lax.shift_right_logical(prefix, 19)

    @plsc.parallel_loop(0, nvec, unroll=8)
    def l1(i):
        z = lax.bitcast_convert_type(row_ref[pl.ds(i * L, L)], jnp.int32)
        m = lax.shift_right_logical(z, 19) == p12
        hi8 = lax.shift_right_logical(z, 11) & 255
        lo4 = lax.shift_right_logical(z, 7) & 15
        plsc.addupdate_scatter(hist_ref, [lo4 * 256 + hi8], ones, mask=m)

    _merge_hist(hist_ref, coarse_ref)
    b_hi, r = _scan_coarse(coarse_ref, 16, r)
    b_lo, r = _scan_fine(hist_ref, b_hi, r)
    _clear(hist_ref, 4096)
    prefix = prefix | (((b_hi << 4) | b_lo) << 7)
    p24 = lax.shift_right_logical(prefix, 7)

    @plsc.parallel_loop(0, nvec, unroll=8)
    def l2(i):
        z = lax.bitcast_convert_type(row_ref[pl.ds(i * L, L)], jnp.int32)
        m = lax.shift_right_logical(z, 7) == p24
        plsc.addupdate_scatter(hist_ref, [z & 127], ones, mask=m)

    b7, _r = _scan_coarse(hist_ref, 8, r)
    _clear(hist_ref, 128)
    t = prefix | b7

    @plsc.parallel_loop(0, nvec, unroll=8)
    def out_body(i):
        zf = row_ref[pl.ds(i * L, L)]
        z = lax.bitcast_convert_type(zf, jnp.int32)
        row_ref[pl.ds(i * L, L)] = jnp.where(z >= t, zf, 0.0)


def kernel(x):
    m, n = x.shape
    nw = 32
    rows_per_w = m // nw
    nvec = n // L
    mesh = plsc.VectorSubcoreMesh(
        core_axis_name="c", subcore_axis_name="s", num_cores=2, num_subcores=16
    )

    @functools.partial(
        pl.kernel,
        out_type=jax.ShapeDtypeStruct((m, n), jnp.float32),
        mesh=mesh,
        scratch_types=[
            [pltpu.VMEM((n,), jnp.float32)] * 3,
            pltpu.VMEM((4096,), jnp.int32),
            pltpu.VMEM((256,), jnp.int32),
            [pltpu.SemaphoreType.DMA] * 3,
            [pltpu.SemaphoreType.DMA] * 3,
        ],
        compiler_params=pltpu.CompilerParams(needs_layout_passes=False),
    )
    def sc_kernel(x_hbm, out_hbm, rows_v, hist_v, coarse_v, in_sems, out_sems):
        wid = lax.axis_index("s") * 2 + lax.axis_index("c")
        base = wid * rows_per_w
        _clear(hist_v, 4096)

        in_copies = {}
        out_copies = {}

        def start_in(rr):
            b = rr % 3
            in_copies[rr] = pltpu.async_copy(
                x_hbm.at[base + rr], rows_v[b], in_sems[b]
            )

        for rr in range(min(3, rows_per_w)):
            start_in(rr)
        for rr in range(rows_per_w):
            b = rr % 3
            in_copies[rr].wait()
            _row_select(rows_v[b], hist_v, coarse_v, nvec)
            nxt = rr + 2
            if rr >= 1 and nxt < rows_per_w:
                out_copies[rr - 1].wait()
                start_in(nxt)
            out_copies[rr] = pltpu.async_copy(
                rows_v[b], out_hbm.at[base + rr], out_sems[b]
            )
        for rr in range(max(0, rows_per_w - 3), rows_per_w):
            out_copies[rr].wait()

    return sc_kernel(x)


# speedup vs baseline: 1.5801x; 1.3586x over previous
import functools

import jax
import jax.numpy as jnp
from jax import lax
from jax.experimental import pallas as pl
from jax.experimental.pallas import tpu as pltpu
from jax.experimental.pallas import tpu_sc as plsc

K = 256
L = 16


def _clear(ref, nwords):
    @plsc.parallel_loop(0, nwords // L, unroll=8)
    def body(i):
        ref[pl.ds(i * L, L)] = jnp.zeros((L,), jnp.int32)


def _merge_hist(hist_ref, coarse_ref):
    zeros = jnp.zeros((L,), jnp.int32)

    @plsc.parallel_loop(0, 256 // L)
    def body(jb):
        acc = zeros
        for lo4 in range(L):
            acc = acc + hist_ref[pl.ds(lo4 * 256 + jb * L, L)]
        coarse_ref[pl.ds(jb * L, L)] = acc


def _scan_coarse(coarse_ref, nvec, r):
    lanes = lax.iota(jnp.int32, L)

    def body(jj, carry):
        suffix, best_b, cge_b, h_b = carry
        j = (nvec - 1) - jj
        v = coarse_ref[pl.ds(j * L, L)]
        rv = lax.rev(v, (0,))
        cs = plsc.cumsum(rv) + suffix
        m = cs >= r
        l = jnp.max(plsc.all_reduce_ffs(m))
        sel = lanes == l
        cs_l = jnp.max(jnp.where(sel, cs, 0))
        rv_l = jnp.max(jnp.where(sel, rv, 0))
        b = j * L + (L - 1) - l
        upd = jnp.logical_and(l < L, best_b < 0)
        best_b = jnp.where(upd, b, best_b)
        cge_b = jnp.where(upd, cs_l, cge_b)
        h_b = jnp.where(upd, rv_l, h_b)
        suffix = suffix + jnp.sum(v)
        return suffix, best_b, cge_b, h_b

    z32 = jnp.int32(0)
    _, best_b, cge_b, h_b = lax.fori_loop(
        0, nvec, body, (z32, jnp.int32(-1), z32, z32)
    )
    return jnp.maximum(best_b, 0), r - (cge_b - h_b)


def _scan_fine(hist_ref, b_hi, r):
    lanes = lax.iota(jnp.int32, L)
    f = plsc.load_gather(hist_ref, [lanes * 256 + b_hi])
    rv = lax.rev(f, (0,))
    cs = plsc.cumsum(rv)
    m = cs >= r
    l = jnp.max(plsc.all_reduce_ffs(m))
    sel = lanes == l
    cs_l = jnp.max(jnp.where(sel, cs, 0))
    rv_l = jnp.max(jnp.where(sel, rv, 0))
    b_lo = jnp.where(l < L, (L - 1) - l, 0)
    return b_lo, r - (cs_l - rv_l)


def _row_select(row_ref, hist_ref, coarse_ref, nvec):
    ones = jnp.ones((L,), jnp.int32)

    def _key(i):
        v = row_ref[pl.ds(i * L, L)]
        return jnp.maximum(lax.bitcast_convert_type(v, jnp.int32), 0)

    @plsc.parallel_loop(0, nvec, unroll=8)
    def l0(i):
        z = _key(i)
        hi8 = lax.shift_right_logical(z, 23)
        lo4 = lax.shift_right_logical(z, 19) & 15
        plsc.addupdate_scatter(hist_ref, [lo4 * 256 + hi8], ones, mask=z > 0)

    _merge_hist(hist_ref, coarse_ref)
    b_hi, r = _scan_coarse(coarse_ref, 16, jnp.int32(K))
    b_lo, r = _scan_fine(hist_ref, b_hi, r)
    _clear(hist_ref, 4096)
    prefix = ((b_hi << 4) | b_lo) << 19
    p12 = lax.shift_right_logical(prefix, 19)

    @plsc.parallel_loop(0, nvec, unroll=8)
    def l1(i):
        z = lax.bitcast_convert_type(row_ref[pl.ds(i * L, L)], jnp.int32)
        m = lax.shift_right_logical(z, 19) == p12
        hi8 = lax.shift_right_logical(z, 11) & 255
        lo4 = lax.shift_right_logical(z, 7) & 15
        plsc.addupdate_scatter(hist_ref, [lo4 * 256 + hi8], ones, mask=m)

    _merge_hist(hist_ref, coarse_ref)
    b_hi, r = _scan_coarse(coarse_ref, 16, r)
    b_lo, r = _scan_fine(hist_ref, b_hi, r)
    _clear(hist_ref, 4096)
    prefix = prefix | (((b_hi << 4) | b_lo) << 7)
    p24 = lax.shift_right_logical(prefix, 7)

    @plsc.parallel_loop(0, nvec, unroll=8)
    def l2(i):
        z = lax.bitcast_convert_type(row_ref[pl.ds(i * L, L)], jnp.int32)
        m = lax.shift_right_logical(z, 7) == p24
        plsc.addupdate_scatter(hist_ref, [z & 127], ones, mask=m)

    b7, _r = _scan_coarse(hist_ref, 8, r)
    _clear(hist_ref, 128)
    t = prefix | b7

    @plsc.parallel_loop(0, nvec, unroll=8)
    def out_body(i):
        zf = row_ref[pl.ds(i * L, L)]
        z = lax.bitcast_convert_type(zf, jnp.int32)
        row_ref[pl.ds(i * L, L)] = jnp.where(z >= t, zf, 0.0)


def kernel(x):
    m, n = x.shape
    nw = 32
    rows_per_w = m // nw
    nvec = n // L
    mesh = plsc.VectorSubcoreMesh(
        core_axis_name="c", subcore_axis_name="s", num_cores=2, num_subcores=16
    )

    @functools.partial(
        pl.kernel,
        out_type=jax.ShapeDtypeStruct((m, n), jnp.float32),
        mesh=mesh,
        scratch_types=[
            [pltpu.VMEM((n,), jnp.float32)] * 3,
            pltpu.VMEM((4096,), jnp.int32),
            pltpu.VMEM((256,), jnp.int32),
            [pltpu.SemaphoreType.DMA] * 3,
            [pltpu.SemaphoreType.DMA] * 3,
        ],
        compiler_params=pltpu.CompilerParams(needs_layout_passes=False),
    )
    def sc_kernel(x_hbm, out_hbm, rows_v, hist_v, coarse_v, in_sems, out_sems):
        wid = lax.axis_index("s") * 2 + lax.axis_index("c")
        base = wid * rows_per_w
        _clear(hist_v, 4096)

        in_copies = {}
        out_copies = {}

        def start_in(rr):
            b = rr % 3
            in_copies[rr] = pltpu.async_copy(
                x_hbm.at[base + rr], rows_v[b], in_sems[b]
            )

        for rr in range(min(3, rows_per_w)):
            start_in(rr)
        for rr in range(rows_per_w):
            b = rr % 3
            in_copies[rr].wait()
            _row_select(rows_v[b], hist_v, coarse_v, nvec)
            nxt = rr + 2
            if rr >= 1 and nxt < rows_per_w:
                out_copies[rr - 1].wait()
                start_in(nxt)
            out_copies[rr] = pltpu.async_copy(
                rows_v[b], out_hbm.at[base + rr], out_sems[b]
            )
        for rr in range(max(0, rows_per_w - 3), rows_per_w):
            out_copies[rr].wait()

    return sc_kernel(x)
